# baseline (speedup 1.0000x reference)
import jax
import jax.numpy as jnp
from jax.experimental import pallas as pl
from jax.experimental.pallas import tpu as pltpu

_B = 1024
_CH = 64
_UNITS = 100000
_T = 2048
_NT = (_UNITS + _T - 1) // _T


def _pass1(x_ref, w_ref, b_ref, t_ref, lse_ref, loss_ref, m_ref, s_ref, p_ref):
    j = pl.program_id(0)

    @pl.when(j == 0)
    def _init():
        m_ref[...] = jnp.full_like(m_ref, -jnp.inf)
        s_ref[...] = jnp.zeros_like(s_ref)
        p_ref[...] = jnp.zeros_like(p_ref)

    logits = (
        jnp.dot(x_ref[...], w_ref[...], preferred_element_type=jnp.float32)
        + b_ref[...]
    )
    col = j * _T + jax.lax.broadcasted_iota(jnp.int32, (_B, _T), 1)
    valid = col < _UNITS
    masked = jnp.where(valid, logits, -jnp.inf)

    m_old = m_ref[...]
    tile_max = jnp.max(masked, axis=1, keepdims=True)
    m_new = jnp.maximum(m_old, tile_max)
    s_new = s_ref[...] * jnp.exp(m_old - m_new) + jnp.sum(
        jnp.exp(masked - m_new), axis=1, keepdims=True
    )
    m_ref[...] = m_new
    s_ref[...] = s_new

    hit = col == t_ref[...]
    p_ref[...] += jnp.sum(jnp.where(hit, logits, 0.0), axis=1, keepdims=True)

    @pl.when(j == _NT - 1)
    def _fini():
        lse = m_new + jnp.log(s_new)
        lse_ref[...] = lse
        loss_ref[...] = jnp.sum(lse - p_ref[...], axis=0, keepdims=True) / _B


def _pass2(x_ref, w_ref, b_ref, lse_ref, out_ref):
    logits = (
        jnp.dot(x_ref[...], w_ref[...], preferred_element_type=jnp.float32)
        + b_ref[...]
    )
    out_ref[...] = jnp.exp(logits - lse_ref[...])


def kernel(logits, targets, kernel, bias):
    x = logits.astype(jnp.float32).reshape(_B, _CH)
    w = kernel
    b2 = bias.reshape(1, _UNITS)
    t2 = targets.reshape(_B, 1)

    lse, loss = pl.pallas_call(
        _pass1,
        grid=(_NT,),
        in_specs=[
            pl.BlockSpec((_B, _CH), lambda j: (0, 0)),
            pl.BlockSpec((_CH, _T), lambda j: (0, j)),
            pl.BlockSpec((1, _T), lambda j: (0, j)),
            pl.BlockSpec((_B, 1), lambda j: (0, 0)),
        ],
        out_specs=[
            pl.BlockSpec((_B, 1), lambda j: (0, 0)),
            pl.BlockSpec((1, 1), lambda j: (0, 0)),
        ],
        out_shape=[
            jax.ShapeDtypeStruct((_B, 1), jnp.float32),
            jax.ShapeDtypeStruct((1, 1), jnp.float32),
        ],
        scratch_shapes=[
            pltpu.VMEM((_B, 1), jnp.float32),
            pltpu.VMEM((_B, 1), jnp.float32),
            pltpu.VMEM((_B, 1), jnp.float32),
        ],
        compiler_params=pltpu.CompilerParams(
            dimension_semantics=("arbitrary",)
        ),
    )(x, w, b2, t2)

    probs = pl.pallas_call(
        _pass2,
        grid=(_NT,),
        in_specs=[
            pl.BlockSpec((_B, _CH), lambda j: (0, 0)),
            pl.BlockSpec((_CH, _T), lambda j: (0, j)),
            pl.BlockSpec((1, _T), lambda j: (0, j)),
            pl.BlockSpec((_B, 1), lambda j: (0, 0)),
        ],
        out_specs=pl.BlockSpec((_B, _T), lambda j: (0, j)),
        out_shape=jax.ShapeDtypeStruct((_B, _UNITS), jnp.float32),
        compiler_params=pltpu.CompilerParams(
            dimension_semantics=("parallel",)
        ),
    )(x, w, b2, lse)

    return probs, loss[0, 0]
